# Initial kernel scaffold; baseline (speedup 1.0000x reference)
import jax
import jax.numpy as jnp
from jax.experimental import pallas as pl


def kernel(inputs, embedding_weight):
    raise NotImplementedError("write your pallas kernel here")


# speedup vs baseline: 1.1032x; 1.1032x over previous
import functools

import jax
import jax.numpy as jnp
from jax import lax
from jax.experimental import pallas as pl
from jax.experimental.pallas import tpu as pltpu
from jax.experimental.pallas import tpu_sc as plsc


def _make_gather(V, D, B):
    info = plsc.get_sparse_core_info()
    NC, NS = info.num_cores, info.num_subcores
    NW = NC * NS
    assert B % NW == 0
    b_per_w = B // NW
    C = 1600
    assert b_per_w % C == 0
    NCH = b_per_w // C

    mesh = plsc.VectorSubcoreMesh(core_axis_name="c", subcore_axis_name="s")

    @functools.partial(
        pl.kernel,
        mesh=mesh,
        out_type=jax.ShapeDtypeStruct((B, D), jnp.float32),
        compiler_params=pltpu.CompilerParams(use_tc_tiling_on_sc=False),
        scratch_types=[
            pltpu.VMEM((C,), jnp.int32),
            pltpu.VMEM((C, D), jnp.float32),
            pltpu.SemaphoreType.DMA,
        ],
    )
    def k(table_hbm, idx_hbm, out_hbm, idx_v, rows_v, sem):
        wid = lax.axis_index("s") * NC + lax.axis_index("c")
        base = wid * b_per_w

        def body(i, carry):
            start = base + i * C
            pltpu.sync_copy(idx_hbm.at[pl.ds(start, C)], idx_v)
            pltpu.async_copy(table_hbm.at[idx_v], rows_v, sem).wait()
            pltpu.sync_copy(rows_v, out_hbm.at[pl.ds(start, C)])
            return carry

        lax.fori_loop(0, NCH, body, 0)

    return k


def kernel(inputs, embedding_weight):
    Bt, T = inputs.shape
    V, D = embedding_weight.shape
    B = Bt * T
    flat_idx = inputs.reshape(B).astype(jnp.int32)
    out = _make_gather(V, D, B)(embedding_weight, flat_idx)
    return out.reshape(Bt, T, D)


# speedup vs baseline: 1.1129x; 1.0088x over previous
import functools

import jax
import jax.numpy as jnp
from jax import lax
from jax.experimental import pallas as pl
from jax.experimental.pallas import tpu as pltpu
from jax.experimental.pallas import tpu_sc as plsc


def _make_gather(V, D, B):
    info = plsc.get_sparse_core_info()
    NC, NS = info.num_cores, info.num_subcores
    NW = NC * NS
    assert B % NW == 0
    b_per_w = B // NW
    C = 1600
    assert b_per_w % (2 * C) == 0
    NCH = b_per_w // C

    mesh = plsc.VectorSubcoreMesh(core_axis_name="c", subcore_axis_name="s")

    @functools.partial(
        pl.kernel,
        mesh=mesh,
        out_type=jax.ShapeDtypeStruct((B, D), jnp.float32),
        compiler_params=pltpu.CompilerParams(use_tc_tiling_on_sc=False),
        scratch_types=[
            pltpu.VMEM((2, C), jnp.int32),
            pltpu.VMEM((2, C, D), jnp.float32),
            pltpu.SemaphoreType.DMA,
            pltpu.SemaphoreType.DMA,
            pltpu.SemaphoreType.DMA,
            pltpu.SemaphoreType.DMA,
        ],
    )
    def k(table_hbm, idx_hbm, out_hbm, idx_v, rows_v, g0, g1, w0, w1):
        wid = lax.axis_index("s") * NC + lax.axis_index("c")
        base = wid * b_per_w

        def idx_load(chunk, buf):
            pltpu.sync_copy(idx_hbm.at[pl.ds(base + chunk * C, C)], idx_v.at[buf])

        def gather_start(buf, sem):
            return pltpu.async_copy(table_hbm.at[idx_v.at[buf]], rows_v.at[buf], sem)

        def write_start(chunk, buf, sem):
            return pltpu.async_copy(
                rows_v.at[buf], out_hbm.at[pl.ds(base + chunk * C, C)], sem
            )

        idx_load(0, 0)
        gather_start(0, g0)

        def body(j, carry):
            a = 2 * j
            b = a + 1
            idx_load(b, 1)
            cp_gb = gather_start(1, g1)
            pltpu.make_async_copy(
                table_hbm.at[idx_v.at[0]], rows_v.at[0], g0
            ).wait()
            cp_wa = write_start(a, 0, w0)
            @pl.when(a + 2 < NCH)
            def _():
                idx_load(a + 2, 0)
            cp_gb.wait()
            cp_wb = write_start(b, 1, w1)
            cp_wa.wait()
            @pl.when(a + 2 < NCH)
            def _():
                gather_start(0, g0)
            cp_wb.wait()
            return carry

        lax.fori_loop(0, NCH // 2, body, 0)

    return k


def kernel(inputs, embedding_weight):
    Bt, T = inputs.shape
    V, D = embedding_weight.shape
    B = Bt * T
    flat_idx = inputs.reshape(B).astype(jnp.int32)
    out = _make_gather(V, D, B)(embedding_weight, flat_idx)
    return out.reshape(Bt, T, D)


# speedup vs baseline: 1.4111x; 1.2680x over previous
import functools

import jax
import jax.numpy as jnp
from jax import lax
from jax.experimental import pallas as pl
from jax.experimental.pallas import tpu as pltpu
from jax.experimental.pallas import tpu_sc as plsc

_WB = 256


def _make_lookup(V, D, T, B):
    info = plsc.get_sparse_core_info()
    NC, NS, L = info.num_cores, info.num_subcores, info.num_lanes
    NW = NC * NS
    G = V * D // 128
    NB = B // _WB
    NU = T * NB
    assert NU % NW == 0
    u_per_w = NU // NW
    ngrp = _WB // L

    mesh = plsc.VectorSubcoreMesh(core_axis_name="c", subcore_axis_name="s")

    @functools.partial(
        pl.kernel,
        mesh=mesh,
        out_type=jax.ShapeDtypeStruct((T, D, B), jnp.float32),
        compiler_params=pltpu.CompilerParams(
            use_tc_tiling_on_sc=True, needs_layout_passes=False
        ),
        scratch_types=[
            pltpu.VMEM((2, _WB // 128, 128), jnp.int32),
            pltpu.VMEM((2, _WB), jnp.int32),
            pltpu.VMEM((_WB,), jnp.int32),
            pltpu.VMEM((2, _WB // 128, 128, 128), jnp.float32),
            pltpu.VMEM((2, D, _WB), jnp.float32),
            pltpu.SemaphoreType.DMA,
            pltpu.SemaphoreType.DMA,
            pltpu.SemaphoreType.DMA,
            pltpu.SemaphoreType.DMA,
        ],
    )
    def k(tab128, inT, outP, gidx, moff, raw, gbuf, cbuf, g0, g1, w0, w1):
        wid = lax.axis_index("s") * NC + lax.axis_index("c")
        gsems = (g0, g1)
        wsems = (w0, w1)
        lane = lax.iota(jnp.int32, L)
        nseg = _WB // 128

        def unit_tb(j):
            u = wid + NW * j
            return u // NB, (u % NB) * _WB

        def load_idx(j, buf):
            t, b0 = unit_tb(j)
            pltpu.sync_copy(inT.at[t, pl.ds(b0, _WB)], raw)
            for kk in range(ngrp):
                v = raw[pl.ds(kk * L, L)]
                g, s = divmod(kk * L, 128)
                gidx[buf, g, pl.ds(s, L)] = jax.lax.shift_right_logical(v, 2)
                moff[buf, pl.ds(kk * L, L)] = (v & 3) * 32

        def gather_start(buf):
            for g in range(nseg):
                pltpu.async_copy(
                    tab128.at[gidx.at[buf, g]], gbuf.at[buf, g], gsems[buf]
                )

        def gather_wait(buf):
            for g in range(nseg):
                pltpu.make_async_copy(
                    tab128.at[gidx.at[buf, g]], gbuf.at[buf, g], gsems[buf]
                ).wait()

        def extract(buf):
            for kk in range(ngrp):
                g, s = divmod(kk * L, 128)
                rows = lane + s
                cols = moff[buf, pl.ds(kk * L, L)]
                for d in range(D):
                    cbuf[buf, d, pl.ds(kk * L, L)] = plsc.load_gather(
                        gbuf.at[buf, g], [rows, cols + d]
                    )

        def write_start(j, buf):
            t, b0 = unit_tb(j)
            return pltpu.async_copy(
                cbuf.at[buf], outP.at[t, :, pl.ds(b0, _WB)], wsems[buf]
            )

        def write_wait(j, buf):
            t, b0 = unit_tb(j)
            pltpu.make_async_copy(
                cbuf.at[buf], outP.at[t, :, pl.ds(b0, _WB)], wsems[buf]
            ).wait()

        load_idx(0, 0)
        gather_start(0)

        def body2(jj, carry):
            a = 2 * jj
            b = a + 1
            load_idx(b, 1)
            gather_start(1)
            gather_wait(0)
            extract(0)
            cw_a = write_start(a, 0)
            @pl.when(a + 2 < u_per_w)
            def _():
                load_idx(a + 2, 0)

            gather_wait(1)
            extract(1)
            cw_b = write_start(b, 1)
            cw_a.wait()

            @pl.when(a + 2 < u_per_w)
            def _():
                gather_start(0)

            cw_b.wait()
            return carry

        lax.fori_loop(0, u_per_w // 2, body2, 0)

    return k


def kernel(inputs, embedding_weight):
    Bt, T = inputs.shape
    V, D = embedding_weight.shape
    tab128 = embedding_weight.reshape(V * D // 128, 128)
    inT = inputs.T
    outP = _make_lookup(V, D, T, Bt)(tab128, inT)
    return outP.transpose(2, 0, 1)


# speedup vs baseline: 1.4338x; 1.0160x over previous
import functools

import jax
import jax.numpy as jnp
from jax import lax
from jax.experimental import pallas as pl
from jax.experimental.pallas import tpu as pltpu
from jax.experimental.pallas import tpu_sc as plsc

_WB = 128
_NBUF = 4


def _make_lookup(V, D, T, B):
    info = plsc.get_sparse_core_info()
    NC, NS, L = info.num_cores, info.num_subcores, info.num_lanes
    NW = NC * NS
    NB = B // _WB
    NU = T * NB
    assert NU % NW == 0
    u_per_w = NU // NW
    assert u_per_w % _NBUF == 0
    ngrp = _WB // L

    mesh = plsc.VectorSubcoreMesh(core_axis_name="c", subcore_axis_name="s")

    @functools.partial(
        pl.kernel,
        mesh=mesh,
        out_type=jax.ShapeDtypeStruct((T, D, B), jnp.float32),
        compiler_params=pltpu.CompilerParams(
            use_tc_tiling_on_sc=True, needs_layout_passes=False
        ),
        scratch_types=[
            pltpu.VMEM((_NBUF, 1, _WB), jnp.int32),
            pltpu.VMEM((_NBUF, _WB), jnp.int32),
            pltpu.VMEM((_WB,), jnp.int32),
            pltpu.VMEM((_NBUF, _WB, 128), jnp.float32),
            pltpu.VMEM((_NBUF, D, _WB), jnp.float32),
        ]
        + [pltpu.SemaphoreType.DMA] * (2 * _NBUF),
    )
    def k(tab128, inT, outP, gidx, moff, raw, gbuf, cbuf, *sems):
        gsems = sems[:_NBUF]
        wsems = sems[_NBUF:]
        wid = lax.axis_index("s") * NC + lax.axis_index("c")
        lane = lax.iota(jnp.int32, L)

        def unit_tb(j):
            u = wid + NW * j
            return u // NB, (u % NB) * _WB

        def load_idx(j, buf):
            t, b0 = unit_tb(j)
            pltpu.sync_copy(inT.at[t, pl.ds(b0, _WB)], raw)
            for kk in range(ngrp):
                v = raw[pl.ds(kk * L, L)]
                gidx[buf, 0, pl.ds(kk * L, L)] = jax.lax.shift_right_logical(
                    v, 2
                )
                moff[buf, pl.ds(kk * L, L)] = (v & 3) * 32

        def gather_start(buf):
            pltpu.async_copy(
                tab128.at[gidx.at[buf, 0]], gbuf.at[buf], gsems[buf]
            )

        def gather_wait(buf):
            pltpu.make_async_copy(
                tab128.at[gidx.at[buf, 0]], gbuf.at[buf], gsems[buf]
            ).wait()

        def extract(buf):
            for kk in range(ngrp):
                rows = lane + kk * L
                cols = moff[buf, pl.ds(kk * L, L)]
                for d in range(D):
                    cbuf[buf, d, pl.ds(kk * L, L)] = plsc.load_gather(
                        gbuf.at[buf], [rows, cols + d]
                    )

        def write_start(j, buf):
            t, b0 = unit_tb(j)
            pltpu.async_copy(
                cbuf.at[buf], outP.at[t, :, pl.ds(b0, _WB)], wsems[buf]
            )

        def write_wait(buf):
            pltpu.make_async_copy(
                cbuf.at[buf], outP.at[0, :, pl.ds(0, _WB)], wsems[buf]
            ).wait()

        for r in range(_NBUF - 1):
            load_idx(r, r)
            gather_start(r)

        def body(jj, carry):
            for r in range(_NBUF):
                u = _NBUF * jj + r
                nxt = u + _NBUF - 1
                nbuf = (r + _NBUF - 1) % _NBUF

                @pl.when(nxt < u_per_w)
                def _():
                    load_idx(nxt, nbuf)
                    gather_start(nbuf)

                gather_wait(r)

                @pl.when(jj > 0)
                def _():
                    write_wait(r)

                extract(r)
                write_start(u, r)
            return carry

        lax.fori_loop(0, u_per_w // _NBUF, body, 0)
        for r in range(_NBUF):
            write_wait(r)

    return k


def kernel(inputs, embedding_weight):
    Bt, T = inputs.shape
    V, D = embedding_weight.shape
    tab128 = embedding_weight.reshape(V * D // 128, 128)
    inT = inputs.T
    outP = _make_lookup(V, D, T, Bt)(tab128, inT)
    return outP.transpose(2, 0, 1)
